# Initial kernel scaffold; baseline (speedup 1.0000x reference)
import jax
import jax.numpy as jnp
from jax.experimental import pallas as pl


def kernel(indices, embed, gru0_kernel, gru0_rec_kernel, gru0_bias, gru1_kernel, gru1_rec_kernel, gru1_bias, proj_W, proj_b):
    raise NotImplementedError("write your pallas kernel here")


# speedup vs baseline: 1.0527x; 1.0527x over previous
import functools

import jax
import jax.numpy as jnp
from jax.experimental import pallas as pl
from jax.experimental.pallas import tpu as pltpu
from jax.experimental.pallas import tpu_sc as plsc

_NUM_CORES = 2
_NUM_SUBCORES = 16
_NUM_WORKERS = _NUM_CORES * _NUM_SUBCORES
_CHUNK = 128


def _sc_gather(table, idx3d):
    workers, chunks, cw = idx3d.shape
    n = workers * chunks * cw
    d = table.shape[1]
    rows_per_w = chunks * cw

    mesh = plsc.VectorSubcoreMesh(core_axis_name="c", subcore_axis_name="s")

    @functools.partial(
        pl.kernel,
        out_type=jax.ShapeDtypeStruct((n, d), table.dtype),
        mesh=mesh,
        scratch_types=[
            pltpu.VMEM((chunks, _CHUNK), jnp.int32),
            pltpu.VMEM((rows_per_w, d), table.dtype),
            pltpu.SemaphoreType.DMA,
        ],
    )
    def gather_kernel(table_hbm, idx_hbm, out_hbm, idx_v, rows_v, sem):
        c = jax.lax.axis_index("c")
        s = jax.lax.axis_index("s")
        wid = s * _NUM_CORES + c
        pltpu.sync_copy(idx_hbm.at[wid], idx_v)
        copies = [
            pltpu.async_copy(
                table_hbm.at[idx_v.at[k]],
                rows_v.at[pl.ds(k * _CHUNK, _CHUNK)],
                sem,
            )
            for k in range(chunks)
        ]
        for cp in copies:
            cp.wait()
        pltpu.sync_copy(rows_v, out_hbm.at[pl.ds(wid * rows_per_w, rows_per_w)])

    return gather_kernel(table, idx3d)


def _gru_stack(x3d, k0, r0, b0, k1, r1, b1):
    T, B, E = x3d.shape
    H = r0.shape[0]

    def body(x_ref, k0_ref, r0_ref, b0_ref, k1_ref, r1_ref, b1_ref, y_ref):
        k0v = k0_ref[...]
        r0v = r0_ref[...]
        k1v = k1_ref[...]
        r1v = r1_ref[...]
        bi0 = b0_ref[0:1, :]
        br0 = b0_ref[1:2, :]
        bi1 = b1_ref[0:1, :]
        br1 = b1_ref[1:2, :]

        def gru_step(h, x_proj, rec, br):
            hp = jnp.dot(h, rec, preferred_element_type=jnp.float32) + br
            z = jax.nn.sigmoid(x_proj[:, :H] + hp[:, :H])
            r = jax.nn.sigmoid(x_proj[:, H:2 * H] + hp[:, H:2 * H])
            hh = jnp.tanh(x_proj[:, 2 * H:] + r * hp[:, 2 * H:])
            return h + (1.0 - z) * (hh - h)

        h0 = jnp.zeros((B, H), jnp.float32)
        h1 = jnp.zeros((B, H), jnp.float32)
        for t in range(T):
            xp0 = jnp.dot(x_ref[t], k0v, preferred_element_type=jnp.float32) + bi0
            h0 = gru_step(h0, xp0, r0v, br0)
            xp1 = jnp.dot(h0, k1v, preferred_element_type=jnp.float32) + bi1
            h1 = gru_step(h1, xp1, r1v, br1)
            y_ref[:, t, :] = h1

    return pl.pallas_call(
        body,
        out_shape=jax.ShapeDtypeStruct((B, T, H), jnp.float32),
    )(x3d, k0, r0, b0, k1, r1, b1)


def _proj(y2d, w, b2d, blk=512):
    n, h = y2d.shape
    v = w.shape[1]

    def body(y_ref, w_ref, b_ref, o_ref):
        o_ref[...] = (
            jnp.dot(y_ref[...], w_ref[...], preferred_element_type=jnp.float32)
            + b_ref[...]
        )

    return pl.pallas_call(
        body,
        grid=(n // blk,),
        in_specs=[
            pl.BlockSpec((blk, h), lambda i: (i, 0)),
            pl.BlockSpec((h, v), lambda i: (0, 0)),
            pl.BlockSpec((1, v), lambda i: (0, 0)),
        ],
        out_specs=pl.BlockSpec((blk, v), lambda i: (i, 0)),
        out_shape=jax.ShapeDtypeStruct((n, v), jnp.float32),
        compiler_params=pltpu.CompilerParams(
            dimension_semantics=("arbitrary",),
        ),
    )(y2d, w, b2d)


def kernel(indices, embed, gru0_kernel, gru0_rec_kernel, gru0_bias,
           gru1_kernel, gru1_rec_kernel, gru1_bias, proj_W, proj_b):
    B, T = indices.shape
    V, E = embed.shape
    H = gru0_rec_kernel.shape[0]

    idx3d = jnp.transpose(indices).reshape(
        _NUM_WORKERS, -1, _CHUNK).astype(jnp.int32)
    x = _sc_gather(embed, idx3d).reshape(T, B, E)
    y = _gru_stack(x, gru0_kernel, gru0_rec_kernel, gru0_bias,
                   gru1_kernel, gru1_rec_kernel, gru1_bias)
    out2d = _proj(y.reshape(B * T, H), proj_W, proj_b.reshape(1, V))
    return out2d.reshape(B, T, V)
